# baseline (speedup 1.0000x reference)
import functools

import jax
import jax.numpy as jnp
from jax import lax
from jax.experimental import pallas as pl
from jax.experimental.pallas import tpu as pltpu
from jax.experimental.pallas import tpu_sc as plsc

_B = 1024 * 200
_D = 232
_NW = 32
_BPW = _B // _NW
_C = 128
_NCHUNK = _BPW // _C

_mesh = plsc.VectorSubcoreMesh(core_axis_name="c", subcore_axis_name="s")


@functools.partial(
    pl.kernel,
    out_type=jax.ShapeDtypeStruct((_B, _D), jnp.float32),
    mesh=_mesh,
    scratch_types=[
        pltpu.VMEM((_NCHUNK, _C), jnp.int32),
        pltpu.VMEM((_C, _D), jnp.float32),
        pltpu.VMEM((_C, _D), jnp.float32),
        pltpu.SemaphoreType.DMA,
        pltpu.SemaphoreType.DMA,
        pltpu.SemaphoreType.DMA,
        pltpu.SemaphoreType.DMA,
    ],
    compiler_params=pltpu.CompilerParams(use_tc_tiling_on_sc=False),
)
def _gather_kernel(x_hbm, w_hbm, out_hbm, idx_v, buf0, buf1,
                   gsem0, gsem1, wsem0, wsem1):
    wid = lax.axis_index("s") * 2 + lax.axis_index("c")
    base = wid * _BPW
    pltpu.sync_copy(x_hbm.at[wid], idx_v)

    bufs = (buf0, buf1)
    gsems = (gsem0, gsem1)
    wsems = (wsem0, wsem1)

    def wait_gather(b):
        pltpu.make_async_copy(w_hbm.at[pl.ds(0, _C)], bufs[b], gsems[b]).wait()

    def wait_wb(b):
        pltpu.make_async_copy(bufs[b], out_hbm.at[pl.ds(base, _C)],
                              wsems[b]).wait()

    pltpu.async_copy(w_hbm.at[idx_v.at[0]], buf0, gsem0)

    @pl.loop(0, _NCHUNK, step=2)
    def _(g):
        for b in range(2):
            j = g + b
            cur, nxt = b, 1 - b
            wait_gather(cur)
            if b == 0:
                pl.when(j > 0)(lambda: wait_wb(nxt))
                pltpu.async_copy(w_hbm.at[idx_v.at[j + 1]], bufs[nxt],
                                 gsems[nxt])
            else:
                wait_wb(nxt)

                @pl.when(j + 1 < _NCHUNK)
                def _():
                    pltpu.async_copy(w_hbm.at[idx_v.at[j + 1]], bufs[nxt],
                                     gsems[nxt])
            pltpu.async_copy(bufs[cur], out_hbm.at[pl.ds(base + j * _C, _C)],
                             wsems[cur])

    wait_wb(1)


def kernel(x, W):
    xf = x.reshape(_NW, _NCHUNK, _C).astype(jnp.int32)
    out = _gather_kernel(xf, W)
    return out.reshape(x.shape[0], x.shape[1], _D)


# speedup vs baseline: 1.4668x; 1.4668x over previous
import functools

import jax
import jax.numpy as jnp
from jax import lax
from jax.experimental import pallas as pl
from jax.experimental.pallas import tpu as pltpu
from jax.experimental.pallas import tpu_sc as plsc

_B = 1024 * 200
_D = 232
_DP = 256
_NW = 32
_BPW = _B // _NW
_C = 80
_NCHUNK = _BPW // _C

_OFFS = [16 * k for k in range(14)] + [_D - 16]

_mesh = plsc.VectorSubcoreMesh(core_axis_name="c", subcore_axis_name="s")


def _gather_body(x_hbm, w_hbm, out_hbm, idx_v, buf, stage, sem):
    wid = lax.axis_index("s") * 2 + lax.axis_index("c")
    base = wid * _BPW
    pltpu.sync_copy(x_hbm.at[wid], idx_v)

    @pl.loop(0, _NCHUNK)
    def _(j):
        pltpu.async_copy(w_hbm.at[idx_v.at[j]], buf, sem).wait()

        @pl.loop(0, _C)
        def _(r):
            for o in _OFFS:
                stage[r, pl.ds(o, 16)] = buf[r, pl.ds(o, 16)]

        pltpu.sync_copy(stage, out_hbm.at[pl.ds(base + j * _C, _C)])


_gather_kernel = pl.kernel(
    _gather_body,
    out_type=jax.ShapeDtypeStruct((_B, _D), jnp.float32),
    mesh=_mesh,
    scratch_types=[
        pltpu.VMEM((_NCHUNK, _C), jnp.int32),
        pltpu.VMEM((_C, _DP), jnp.float32),
        pltpu.VMEM((_C, _D), jnp.float32),
        pltpu.SemaphoreType.DMA,
    ],
    compiler_params=pltpu.CompilerParams(use_tc_tiling_on_sc=True),
)


def kernel(x, W):
    xf = x.reshape(_NW, _NCHUNK, _C).astype(jnp.int32)
    wp = jnp.pad(W, ((0, 0), (0, _DP - _D)))
    out = _gather_kernel(xf, wp)
    return out.reshape(x.shape[0], x.shape[1], _D)


# speedup vs baseline: 1.8383x; 1.2533x over previous
import functools

import jax
import jax.numpy as jnp
from jax import lax
from jax.experimental import pallas as pl
from jax.experimental.pallas import tpu as pltpu
from jax.experimental.pallas import tpu_sc as plsc

_B = 1024 * 200
_D = 232
_DP = 256
_NW = 32
_BPW = _B // _NW
_C = 80
_NCHUNK = _BPW // _C

_OFFS = [16 * k for k in range(14)] + [_D - 16]

_mesh = plsc.VectorSubcoreMesh(core_axis_name="c", subcore_axis_name="s")


def _gather_body(x_hbm, w_hbm, out_hbm, idx_v, bufs, stages, gsems, wsems):
    wid = lax.axis_index("s") * 2 + lax.axis_index("c")
    base = wid * _BPW
    pltpu.sync_copy(x_hbm.at[wid], idx_v)

    def wait_gather(b):
        pltpu.make_async_copy(w_hbm.at[pl.ds(0, _C)], bufs[b],
                              gsems[b]).wait()

    def wait_wb(b):
        pltpu.make_async_copy(stages[b], out_hbm.at[pl.ds(base, _C)],
                              wsems[b]).wait()

    def compact(b):
        @pl.loop(0, _C)
        def _(r):
            for o in _OFFS:
                stages[b][r, pl.ds(o, 16)] = bufs[b][r, pl.ds(o, 16)]

    pltpu.async_copy(w_hbm.at[idx_v.at[0]], bufs[0], gsems[0])

    @pl.loop(0, _NCHUNK, step=2)
    def _(g):
        for b in range(2):
            j = g + b
            cur, nxt = b, 1 - b
            wait_gather(cur)
            if b == 0:
                pltpu.async_copy(w_hbm.at[idx_v.at[j + 1]], bufs[nxt],
                                 gsems[nxt])
            else:
                @pl.when(j + 1 < _NCHUNK)
                def _():
                    pltpu.async_copy(w_hbm.at[idx_v.at[j + 1]], bufs[nxt],
                                     gsems[nxt])
            pl.when(j >= 2)(lambda bb=cur: wait_wb(bb))
            compact(cur)
            pltpu.async_copy(stages[cur],
                             out_hbm.at[pl.ds(base + j * _C, _C)],
                             wsems[cur])

    wait_wb(0)
    wait_wb(1)


_gather_kernel = pl.kernel(
    _gather_body,
    out_type=jax.ShapeDtypeStruct((_B, _D), jnp.float32),
    mesh=_mesh,
    scratch_types=[
        pltpu.VMEM((_NCHUNK, _C), jnp.int32),
        [pltpu.VMEM((_C, _DP), jnp.float32) for _ in range(2)],
        [pltpu.VMEM((_C, _D), jnp.float32) for _ in range(2)],
        [pltpu.SemaphoreType.DMA for _ in range(2)],
        [pltpu.SemaphoreType.DMA for _ in range(2)],
    ],
    compiler_params=pltpu.CompilerParams(use_tc_tiling_on_sc=True),
)


def kernel(x, W):
    xf = x.reshape(_NW, _NCHUNK, _C).astype(jnp.int32)
    wp = jnp.pad(W, ((0, 0), (0, _DP - _D)))
    out = _gather_kernel(xf, wp)
    return out.reshape(x.shape[0], x.shape[1], _D)
